# baseline (speedup 1.0000x reference)
import functools

import jax
import jax.numpy as jnp
from jax import lax
from jax.experimental import pallas as pl
from jax.experimental.pallas import tpu as pltpu
from jax.experimental.pallas import tpu_sc as plsc

VOCAB = 1_000_000
EMBED = 32
BATCH = 4096
LABELS = 50

NC = 2
NS = 16
NW = NC * NS

ROWS_PER_W = BATCH // NW
LINES = VOCAB // 4
IDXW = 100
IDX_ROWS = BATCH * LABELS // IDXW
IDX_ROWS_PER_W = IDX_ROWS // NW
K = 4
CH = K * IDXW
CHUNKS = IDX_ROWS_PER_W // K
BR = CH // LABELS
LPAD = 64
ZPOS = CH * 128

_DIMNUM = lax.GatherDimensionNumbers(
    offset_dims=(), collapsed_slice_dims=(0,), start_index_map=(0,))


def _splat(v, i):
    idx = jnp.full((16, 1), i, dtype=jnp.int32)
    return lax.gather(v, idx, _DIMNUM, slice_sizes=(1,),
                      mode=lax.GatherScatterMode.PROMISE_IN_BOUNDS)


def _sc_body(emb_hbm, lines_hbm, pos_hbm, out_hbm, idx_v, pos_v, buf_v, out_v,
             sem):
    wid = lax.axis_index("s") * NC + lax.axis_index("c")
    zero = jnp.zeros((16,), jnp.float32)
    for h in range(8):
        buf_v[CH, 16 * h:16 * h + 16] = zero
    lane = lax.iota(jnp.int32, 16)

    def chunk(c, carry):
        pltpu.sync_copy(lines_hbm.at[pl.ds(wid * IDX_ROWS_PER_W + c * K, K)],
                        idx_v)
        pltpu.sync_copy(pos_hbm.at[pl.ds(wid * ROWS_PER_W + c * BR, BR)],
                        pos_v)
        copies = [
            pltpu.async_copy(
                emb_hbm.at[idx_v.at[j]],
                buf_v.at[pl.ds(j * IDXW, IDXW)],
                sem,
            )
            for j in range(K)
        ]
        for cp in copies:
            cp.wait()

        def body(r, carry2):
            a0 = zero
            a1 = zero
            for g in range(LPAD // 16):
                bv = pos_v[r, 16 * g:16 * g + 16]
                for i in range(16):
                    b = _splat(bv, i)
                    rowv = lax.shift_right_logical(b, 7)
                    colv = (b & 127) + lane
                    a0 = a0 + plsc.load_gather(buf_v, [rowv, colv])
                    a1 = a1 + plsc.load_gather(buf_v, [rowv, colv + 16])
            out_v[r, 0:16] = a0
            out_v[r, 16:32] = a1
            return carry2

        lax.fori_loop(0, BR, body, 0)
        pltpu.sync_copy(out_v, out_hbm.at[pl.ds(wid * ROWS_PER_W + c * BR, BR)])
        return carry

    lax.fori_loop(0, CHUNKS, chunk, 0)


@jax.jit
def _run(inputs, emb):
    emb2 = emb.reshape(LINES, 128)
    flat = inputs.reshape(-1).astype(jnp.int32)
    lines = (flat // 4).reshape(IDX_ROWS, IDXW)
    bufpos = (jnp.arange(BATCH * LABELS, dtype=jnp.int32) % CH) // LABELS
    pos = (bufpos * LABELS + jnp.arange(BATCH * LABELS, dtype=jnp.int32)
           % LABELS) * 128 + (flat % 4) * 32
    pos = pos.reshape(BATCH, LABELS)
    pad = jnp.full((BATCH, LPAD - LABELS), ZPOS, dtype=jnp.int32)
    pos = jnp.concatenate([pos, pad], axis=1)

    mesh = plsc.VectorSubcoreMesh(core_axis_name="c", subcore_axis_name="s")
    f = functools.partial(
        pl.kernel,
        mesh=mesh,
        compiler_params=pltpu.CompilerParams(needs_layout_passes=False),
        out_type=jax.ShapeDtypeStruct((BATCH, EMBED), jnp.float32),
        scratch_types=[
            pltpu.VMEM((K, IDXW), jnp.int32),
            pltpu.VMEM((BR, LPAD), jnp.int32),
            pltpu.VMEM((CH + 1, 128), jnp.float32),
            pltpu.VMEM((BR, EMBED), jnp.float32),
            pltpu.SemaphoreType.DMA,
        ],
    )(_sc_body)
    return f(emb2, lines, pos)


def kernel(inputs, emb):
    return _run(inputs, emb)
